# baseline (speedup 1.0000x reference)
import jax
import jax.numpy as jnp
from jax.experimental import pallas as pl

_MAX_LEN = 8192
_S_BLK = 2048
_B_BLK = 1


def _add_body(x_ref, emb_ref, out_ref):
    out_ref[...] = x_ref[...] + emb_ref[...][None, :, :]


def kernel(x, rel_pos_emb):
    batch, seq_len, d_model = x.shape
    n_blocks = seq_len // _S_BLK
    emb_off = _MAX_LEN // _S_BLK
    return pl.pallas_call(
        _add_body,
        grid=(n_blocks, batch // _B_BLK),
        in_specs=[
            pl.BlockSpec((_B_BLK, _S_BLK, d_model), lambda j, b: (b, j, 0)),
            pl.BlockSpec((_S_BLK, d_model), lambda j, b: (emb_off + j, 0)),
        ],
        out_specs=pl.BlockSpec((_B_BLK, _S_BLK, d_model), lambda j, b: (b, j, 0)),
        out_shape=jax.ShapeDtypeStruct((batch, seq_len, d_model), x.dtype),
    )(x, rel_pos_emb)


# speedup vs baseline: 1.0037x; 1.0037x over previous
import jax
import jax.numpy as jnp
from jax.experimental import pallas as pl

_MAX_LEN = 8192
_S_BLK = 1024
_B_BLK = 2


def _add_body(x_ref, emb_ref, out_ref):
    out_ref[...] = x_ref[...] + emb_ref[...][None, :, :]


def kernel(x, rel_pos_emb):
    batch, seq_len, d_model = x.shape
    n_blocks = seq_len // _S_BLK
    emb_off = _MAX_LEN // _S_BLK
    return pl.pallas_call(
        _add_body,
        grid=(n_blocks, batch // _B_BLK),
        in_specs=[
            pl.BlockSpec((_B_BLK, _S_BLK, d_model), lambda j, b: (b, j, 0)),
            pl.BlockSpec((_S_BLK, d_model), lambda j, b: (emb_off + j, 0)),
        ],
        out_specs=pl.BlockSpec((_B_BLK, _S_BLK, d_model), lambda j, b: (b, j, 0)),
        out_shape=jax.ShapeDtypeStruct((batch, seq_len, d_model), x.dtype),
    )(x, rel_pos_emb)
